# baseline (speedup 1.0000x reference)
import jax
import jax.numpy as jnp
from jax.experimental import pallas as pl

N_GRID_C = 324
N_MESH_C = 81


def _silu(x):
    return x * jax.lax.logistic(x)


def _ln(x, lnp):
    s, b = lnp
    j = jnp.full((128, 128), 1.0 / 128.0, dtype=jnp.float32)
    mu = jnp.dot(x, j, preferred_element_type=jnp.float32)
    m2 = jnp.dot(x * x, j, preferred_element_type=jnp.float32)
    inv = jax.lax.rsqrt(m2 - mu * mu + 1e-5)
    return (x - mu) * inv * s[:] + b[:]


def _mlp(p, x):
    layers = p["layers"]
    n = len(layers)
    for li, (Wr, br) in enumerate(layers):
        x = _mm(x, Wr[:]) + br[:]
        if li < n - 1:
            x = _silu(x)
    if "ln" in p:
        x = _ln(x, p["ln"])
    return x


def _tail(players, z, pln):
    for li in (1, 2):
        Wr, br = players[li]
        z = _mm(z, Wr[:]) + br[:]
        if li < 2:
            z = _silu(z)
    return _ln(z, pln)


def _rep3(a):
    n, d = a.shape
    return jnp.broadcast_to(a[:, None, :], (n, 3, d)).reshape(n * 3, d)


def _sum3(a):
    e, d = a.shape
    return jnp.sum(a.reshape(e // 3, 3, d), axis=1)


def _mm(a, b):
    return jnp.dot(
        a.astype(jnp.bfloat16),
        b.astype(jnp.bfloat16),
        preferred_element_type=jnp.float32,
    )


def _mp_block(p, h_src, h_dst, e, gather_src, gather_dst, scatter, zero_dst):
    pe = p["edge"]["layers"]
    W0r, b0r = pe[0]
    z = gather_src(_mm(h_src, W0r[0:128]))
    z = z + _mm(e, W0r[256:384]) + b0r[:]
    if not zero_dst:
        z = z + gather_dst(_mm(h_dst, W0r[128:256]))
    z = _silu(z)
    e_new = e + _tail(pe, z, p["edge"]["ln"])

    agg = scatter(e_new)

    pn = p["node"]["layers"]
    V0r, c0r = pn[0]
    y = _mm(agg, V0r[128:256]) + c0r[:]
    if not zero_dst:
        y = y + _mm(h_dst, V0r[0:128])
    y = _silu(y)
    y = _tail(pn, y, p["node"]["ln"])
    h_new = y if zero_dst else h_dst + y
    return h_new, e_new


def _onehot(col_ref, n):
    e = col_ref.shape[0]
    ids = jax.lax.broadcasted_iota(jnp.int32, (e, n), 1)
    return (ids == col_ref[:]).astype(jnp.float32)


def _onehot_t(row_ref, n):
    e = row_ref.shape[1]
    ids = jax.lax.broadcasted_iota(jnp.int32, (n, e), 0)
    return (ids == row_ref[:]).astype(jnp.float32)


def _forecaster(P, x, attrs, oh):
    h_g = _mlp(P["enc_node"], x)
    e = _mlp(P["enc_edge"], attrs["g2m"][:])
    h_m, e = _mp_block(
        P["enc_blk"], h_g, None, e,
        lambda t: _mm(oh["g2m_src"], t), None,
        lambda t: _mm(oh["g2m_dst_t"], t),
        zero_dst=True,
    )
    em = _mlp(P["m2m_edge"], attrs["m2m"][:])
    for bi in range(3):
        h_m, em = _mp_block(
            P["proc"][bi], h_m, h_m, em,
            lambda t: _mm(oh["m2m_src"], t),
            lambda t: _mm(oh["m2m_dst"], t),
            lambda t: _mm(oh["m2m_dst_t"], t),
            zero_dst=False,
        )
    ed = _mlp(P["dec_edge"], attrs["m2g"][:])
    h_g, ed = _mp_block(
        P["dec_blk"], h_m, h_g, ed,
        lambda t: _mm(oh["m2g_src"], t),
        lambda t: _mm(oh["m2g_dst"], t),
        lambda t: _mm(oh["m2g_dst_t"], t),
        zero_dst=False,
    )
    return x + _mlp(P["dec_out"], h_g)


def _fc_kernel(x_ref, ps_ref, attrs, idx, P1, P2, P3, out_ref):
    oh = {
        "g2m_src": _onehot(idx["g2m_src_c"], N_GRID_C),
        "g2m_dst_t": _onehot_t(idx["g2m_dst_r"], N_MESH_C),
        "m2m_src": _onehot(idx["m2m_src_c"], N_MESH_C),
        "m2m_dst": _onehot(idx["m2m_dst_c"], N_MESH_C),
        "m2m_dst_t": _onehot_t(idx["m2m_dst_r"], N_MESH_C),
        "m2g_src": _onehot(idx["m2g_src_c"], N_MESH_C),
        "m2g_dst": _onehot(idx["m2g_dst_c"], N_GRID_C),
        "m2g_dst_t": _onehot_t(idx["m2g_dst_r"], N_GRID_C),
    }
    acc = None
    for mi, P in enumerate((P1, P2, P3)):
        x = x_ref[0, mi]
        o = _forecaster(P, x, attrs, oh)
        w = ps_ref[mi : mi + 1, :]
        acc = o * w if acc is None else acc + o * w
    out_ref[:] = acc


def kernel(features, params1, params2, params3, p1, p2, p3, g2m_attr, m2m_attr,
           m2g_attr, g2m_src, g2m_dst, m2m_src, m2m_dst, m2g_src, m2g_dst):
    ps = jnp.stack([p1, p2, p3]).astype(jnp.float32).reshape(3, 1)
    attrs = {"g2m": g2m_attr, "m2m": m2m_attr, "m2g": m2g_attr}
    idx = {
        "g2m_src_c": g2m_src.reshape(-1, 1),
        "g2m_dst_r": g2m_dst.reshape(1, -1),
        "m2m_src_c": m2m_src.reshape(-1, 1),
        "m2m_dst_c": m2m_dst.reshape(-1, 1),
        "m2m_dst_r": m2m_dst.reshape(1, -1),
        "m2g_src_c": m2g_src.reshape(-1, 1),
        "m2g_dst_c": m2g_dst.reshape(-1, 1),
        "m2g_dst_r": m2g_dst.reshape(1, -1),
    }
    out = pl.pallas_call(
        _fc_kernel,
        out_shape=jax.ShapeDtypeStruct((N_GRID_C, 42), jnp.float32),
    )(features, ps, attrs, idx, params1, params2, params3)
    return out[None]


# speedup vs baseline: 1.2115x; 1.2115x over previous
import jax
import jax.numpy as jnp
from jax.experimental import pallas as pl

N_GRID_C = 324
N_MESH_C = 81


def _silu(x):
    return x * jax.lax.logistic(x)


def _ln(x, lnp):
    s, b = lnp
    mu = jnp.mean(x, axis=-1, keepdims=True)
    var = jnp.mean(jnp.square(x - mu), axis=-1, keepdims=True)
    return (x - mu) * jax.lax.rsqrt(var + 1e-5) * s[:] + b[:]


def _mlp(p, x):
    layers = p["layers"]
    n = len(layers)
    for li, (Wr, br) in enumerate(layers):
        x = _mm(x, Wr[:]) + br[:]
        if li < n - 1:
            x = _silu(x)
    if "ln" in p:
        x = _ln(x, p["ln"])
    return x


def _tail(players, z, pln):
    for li in (1, 2):
        Wr, br = players[li]
        z = _mm(z, Wr[:]) + br[:]
        if li < 2:
            z = _silu(z)
    return _ln(z, pln)


def _rep3(a):
    n, d = a.shape
    return jnp.broadcast_to(a[:, None, :], (n, 3, d)).reshape(n * 3, d)


def _sum3(a):
    e, d = a.shape
    return jnp.sum(a.reshape(e // 3, 3, d), axis=1)


def _mm(a, b):
    return jnp.dot(a, b, preferred_element_type=jnp.float32)


def _mp_block(p, h_src, h_dst, e, gather_src, gather_dst, scatter, zero_dst):
    pe = p["edge"]["layers"]
    W0r, b0r = pe[0]
    z = gather_src(_mm(h_src, W0r[0:128]))
    z = z + _mm(e, W0r[256:384]) + b0r[:]
    if not zero_dst:
        z = z + gather_dst(_mm(h_dst, W0r[128:256]))
    z = _silu(z)
    e_new = e + _tail(pe, z, p["edge"]["ln"])

    agg = scatter(e_new)

    pn = p["node"]["layers"]
    V0r, c0r = pn[0]
    y = _mm(agg, V0r[128:256]) + c0r[:]
    if not zero_dst:
        y = y + _mm(h_dst, V0r[0:128])
    y = _silu(y)
    y = _tail(pn, y, p["node"]["ln"])
    h_new = y if zero_dst else h_dst + y
    return h_new, e_new


def _onehot(col_ref, n):
    e = col_ref.shape[0]
    ids = jax.lax.broadcasted_iota(jnp.int32, (e, n), 1)
    return (ids == col_ref[:]).astype(jnp.float32)


def _onehot_t(row_ref, n):
    e = row_ref.shape[1]
    ids = jax.lax.broadcasted_iota(jnp.int32, (n, e), 0)
    return (ids == row_ref[:]).astype(jnp.float32)


def _forecasters_lockstep(Ps, xs, attrs, oh):
    n = len(Ps)
    g2m = attrs["g2m"][:]
    m2m = attrs["m2m"][:]
    m2g = attrs["m2g"][:]
    h_g = [_mlp(Ps[i]["enc_node"], xs[i]) for i in range(n)]
    e = [_mlp(Ps[i]["enc_edge"], g2m) for i in range(n)]
    h_m = [None] * n
    for i in range(n):
        h_m[i], e[i] = _mp_block(
            Ps[i]["enc_blk"], h_g[i], None, e[i],
            lambda t: _mm(oh["g2m_src"], t), None,
            lambda t: _mm(oh["g2m_dst_t"], t),
            zero_dst=True,
        )
    em = [_mlp(Ps[i]["m2m_edge"], m2m) for i in range(n)]
    for bi in range(3):
        for i in range(n):
            h_m[i], em[i] = _mp_block(
                Ps[i]["proc"][bi], h_m[i], h_m[i], em[i],
                lambda t: _mm(oh["m2m_src"], t),
                lambda t: _mm(oh["m2m_dst"], t),
                lambda t: _mm(oh["m2m_dst_t"], t),
                zero_dst=False,
            )
    ed = [_mlp(Ps[i]["dec_edge"], m2g) for i in range(n)]
    for i in range(n):
        h_g[i], ed[i] = _mp_block(
            Ps[i]["dec_blk"], h_m[i], h_g[i], ed[i],
            lambda t: _mm(oh["m2g_src"], t),
            lambda t: _mm(oh["m2g_dst"], t),
            lambda t: _mm(oh["m2g_dst_t"], t),
            zero_dst=False,
        )
    return [xs[i] + _mlp(Ps[i]["dec_out"], h_g[i]) for i in range(n)]


def _fc_kernel(x_ref, ps_ref, attrs, idx, P1, P2, P3, out_ref):
    oh = {
        "g2m_src": _onehot(idx["g2m_src_c"], N_GRID_C),
        "g2m_dst_t": _onehot_t(idx["g2m_dst_r"], N_MESH_C),
        "m2m_src": _onehot(idx["m2m_src_c"], N_MESH_C),
        "m2m_dst": _onehot(idx["m2m_dst_c"], N_MESH_C),
        "m2m_dst_t": _onehot_t(idx["m2m_dst_r"], N_MESH_C),
        "m2g_src": _onehot(idx["m2g_src_c"], N_MESH_C),
        "m2g_dst": _onehot(idx["m2g_dst_c"], N_GRID_C),
        "m2g_dst_t": _onehot_t(idx["m2g_dst_r"], N_GRID_C),
    }
    xs = [x_ref[0, mi] for mi in range(3)]
    outs = _forecasters_lockstep((P1, P2, P3), xs, attrs, oh)
    acc = None
    for mi in range(3):
        w = ps_ref[mi : mi + 1, :]
        acc = outs[mi] * w if acc is None else acc + outs[mi] * w
    out_ref[:] = acc


def kernel(features, params1, params2, params3, p1, p2, p3, g2m_attr, m2m_attr,
           m2g_attr, g2m_src, g2m_dst, m2m_src, m2m_dst, m2g_src, m2g_dst):
    ps = jnp.stack([p1, p2, p3]).astype(jnp.float32).reshape(3, 1)
    attrs = {"g2m": g2m_attr, "m2m": m2m_attr, "m2g": m2g_attr}
    idx = {
        "g2m_src_c": g2m_src.reshape(-1, 1),
        "g2m_dst_r": g2m_dst.reshape(1, -1),
        "m2m_src_c": m2m_src.reshape(-1, 1),
        "m2m_dst_c": m2m_dst.reshape(-1, 1),
        "m2m_dst_r": m2m_dst.reshape(1, -1),
        "m2g_src_c": m2g_src.reshape(-1, 1),
        "m2g_dst_c": m2g_dst.reshape(-1, 1),
        "m2g_dst_r": m2g_dst.reshape(1, -1),
    }
    out = pl.pallas_call(
        _fc_kernel,
        out_shape=jax.ShapeDtypeStruct((N_GRID_C, 42), jnp.float32),
    )(features, ps, attrs, idx, params1, params2, params3)
    return out[None]
